# Initial kernel scaffold; baseline (speedup 1.0000x reference)
import jax
import jax.numpy as jnp
from jax.experimental import pallas as pl


def kernel(x, edge_index, edge_attr, ground_node, node_subnode_index, subgraph_edge_index, subnode_node_index, batch, params):
    raise NotImplementedError("write your pallas kernel here")


# speedup vs baseline: 13.4594x; 13.4594x over previous
import functools

import jax
import jax.numpy as jnp
from jax import lax
from jax.experimental import pallas as pl
from jax.experimental.pallas import tpu as pltpu
from jax.experimental.pallas import tpu_sc as plsc

N = 10000
E = 160000
D = 256
H = 256
HH = 128
OUT = 128
G = 64
DEPTH = 2

NC = 2
NS = 16
CH = 80
CPT = E // NS // CH
RPT = N // NS
DW = 16

BLK = 2000
NB = N // BLK

_f32 = jnp.float32


def _embed_body(x_ref, w_ref, b_ref, o_ref):
    o_ref[...] = (
        jnp.dot(x_ref[...], w_ref[...], preferred_element_type=_f32)
        + b_ref[...]
    )


def _pre_body(h_ref, w_ref, deg_ref, o_ref):
    t = jnp.dot(h_ref[...], w_ref[...], preferred_element_type=_f32)
    t = t * lax.rsqrt(deg_ref[...])
    o_ref[0] = t[:, :HH]
    o_ref[1] = t[:, HH:]


def _post_body(s_ref, h_ref, gn_ref, deg_ref, b_ref, o_ref, *, upg):
    sblk = jnp.concatenate([s_ref[0], s_ref[1]], axis=-1)
    hn = jnp.maximum(sblk * lax.rsqrt(deg_ref[...]) + b_ref[...], 0.0)
    m = gn_ref[...] > 0.0
    if upg:
        o_ref[...] = jnp.where(m, hn, h_ref[...])
    else:
        o_ref[...] = jnp.where(m, h_ref[...], hn)


def _pool_body(h_ref, seg_ref, wh_ref, bh_ref, o_ref, acc_ref):
    j = pl.program_id(0)

    @pl.when(j == 0)
    def _init():
        acc_ref[...] = jnp.zeros_like(acc_ref)

    onehot = (
        seg_ref[...][:, 0][None, :]
        == lax.broadcasted_iota(jnp.int32, (G, BLK), 0)
    ).astype(_f32)
    acc_ref[...] = acc_ref[...] + jnp.dot(
        onehot, h_ref[...], preferred_element_type=_f32
    )

    @pl.when(j == pl.num_programs(0) - 1)
    def _fin():
        o_ref[...] = (
            jnp.dot(acc_ref[...], wh_ref[...], preferred_element_type=_f32)
            + bh_ref[...]
        )


def _embed(x, w, b):
    return pl.pallas_call(
        _embed_body,
        grid=(NB,),
        in_specs=[
            pl.BlockSpec((BLK, D), lambda j: (j, 0)),
            pl.BlockSpec((D, H), lambda j: (0, 0)),
            pl.BlockSpec((1, H), lambda j: (0, 0)),
        ],
        out_specs=pl.BlockSpec((BLK, H), lambda j: (j, 0)),
        out_shape=jax.ShapeDtypeStruct((N, H), _f32),
    )(x, w, b)


def _pre(h, w, deg):
    return pl.pallas_call(
        _pre_body,
        grid=(NB,),
        in_specs=[
            pl.BlockSpec((BLK, H), lambda j: (j, 0)),
            pl.BlockSpec((H, H), lambda j: (0, 0)),
            pl.BlockSpec((BLK, 1), lambda j: (j, 0)),
        ],
        out_specs=pl.BlockSpec((2, BLK, HH), lambda j: (0, j, 0)),
        out_shape=jax.ShapeDtypeStruct((2, N, HH), _f32),
    )(h, w, deg)


def _post(s, h, gn, deg, b, upg):
    return pl.pallas_call(
        functools.partial(_post_body, upg=upg),
        grid=(NB,),
        in_specs=[
            pl.BlockSpec((2, BLK, HH), lambda j: (0, j, 0)),
            pl.BlockSpec((BLK, H), lambda j: (j, 0)),
            pl.BlockSpec((BLK, 1), lambda j: (j, 0)),
            pl.BlockSpec((BLK, 1), lambda j: (j, 0)),
            pl.BlockSpec((1, H), lambda j: (0, 0)),
        ],
        out_specs=pl.BlockSpec((BLK, H), lambda j: (j, 0)),
        out_shape=jax.ShapeDtypeStruct((N, H), _f32),
    )(s, h, gn, deg, b)


def _pool_head(h, seg, wh, bh):
    return pl.pallas_call(
        _pool_body,
        grid=(NB,),
        in_specs=[
            pl.BlockSpec((BLK, H), lambda j: (j, 0)),
            pl.BlockSpec((BLK, 1), lambda j: (j, 0)),
            pl.BlockSpec((H, OUT), lambda j: (0, 0)),
            pl.BlockSpec((1, OUT), lambda j: (0, 0)),
        ],
        out_specs=pl.BlockSpec((G, OUT), lambda j: (0, 0)),
        out_shape=jax.ShapeDtypeStruct((G, OUT), _f32),
        scratch_shapes=[pltpu.VMEM((G, H), _f32)],
    )(h, seg, wh, bh)


_MESH = plsc.VectorSubcoreMesh(core_axis_name="c", subcore_axis_name="s")
_OCH = 125


def _deg_body(dst_hbm, deg_hbm, acc0, acc1, ones_v, idx_v, sem):
    c = lax.axis_index("c")
    s = lax.axis_index("s")

    for i in range(_OCH):
        ones_v[i] = jnp.ones((DW,), _f32)

    for a in (acc0, acc1):
        for i in range(RPT // _OCH):
            pltpu.sync_copy(ones_v, a.at[pl.ds(s * RPT + i * _OCH, _OCH)])
    plsc.subcore_barrier()

    for k, a in ((0, acc0), (1, acc1)):
        g = 2 * c + k
        pltpu.sync_copy(dst_hbm.at[g * NS + s], idx_v)

        def _fire(r, _):
            pltpu.async_copy(
                ones_v.at[pl.ds(0, CH)], a.at[idx_v.at[r]], sem, add=True
            )
            return 0

        lax.fori_loop(0, CPT, _fire, 0)

        def _drain(r, _):
            pltpu.make_async_copy(
                ones_v.at[pl.ds(0, CH)], a.at[idx_v.at[r]], sem
            ).wait()
            return 0

        lax.fori_loop(0, CPT, _drain, 0)
    plsc.subcore_barrier()

    for k, a in ((0, acc0), (1, acc1)):
        pltpu.sync_copy(
            a.at[pl.ds(s * RPT, RPT)], deg_hbm.at[(2 * c + k) * NS + s]
        )


@functools.partial(
    pl.kernel,
    out_type=jax.ShapeDtypeStruct((4 * NS, RPT, DW), _f32),
    mesh=_MESH,
    scratch_types=[
        pltpu.VMEM_SHARED((N, DW), _f32),
        pltpu.VMEM_SHARED((N, DW), _f32),
        pltpu.VMEM((_OCH, DW), _f32),
        pltpu.VMEM((CPT, CH), jnp.int32),
        pltpu.SemaphoreType.DMA,
    ],
)
def _degrees_sc(dst_hbm, deg_hbm, acc0, acc1, ones_v, idx_v, sem):
    _deg_body(dst_hbm, deg_hbm, acc0, acc1, ones_v, idx_v, sem)


def _conv_body(t2_hbm, t23_hbm, src_hbm, dst_hbm, s_hbm, acc, idxs_v, idxd_v,
               rows_v, sem_g0, sem_g1, sem_s0, sem_s1):
    c = lax.axis_index("c")
    s = lax.axis_index("s")
    rbase = s * RPT
    half = c * N
    ebase = s * (CPT * CH)

    pltpu.sync_copy(t23_hbm.at[c * NS + s], acc.at[pl.ds(rbase, RPT)])

    pltpu.sync_copy(dst_hbm.at[s], idxd_v)
    plsc.subcore_barrier()

    gsems = (sem_g0, sem_g1)
    ssems = (sem_s0, sem_s1)

    def _src_start(r, b):
        pltpu.async_copy(
            src_hbm.at[pl.ds(ebase + r * CH, CH)], idxs_v.at[b], ssems[b]
        )

    def _src_wait_shift(r, b):
        pltpu.make_async_copy(
            src_hbm.at[pl.ds(ebase + r * CH, CH)], idxs_v.at[b], ssems[b]
        ).wait()
        for t in range(CH // 16):
            sl = pl.ds(t * 16, 16)
            idxs_v[b, sl] = idxs_v[b, sl] + half

    def _g_start(r, b):
        pltpu.async_copy(t2_hbm.at[idxs_v.at[b]], rows_v.at[b], gsems[b])

    def _g_wait(r, b):
        pltpu.make_async_copy(
            t2_hbm.at[idxs_v.at[b]], rows_v.at[b], gsems[b]
        ).wait()

    def _scatter(r, b):
        pltpu.sync_copy(rows_v.at[b], acc.at[idxd_v.at[r]], add=True)

    _src_start(0, 0)
    _src_wait_shift(0, 0)
    _g_start(0, 0)
    _src_start(1, 1)

    def _step(r, b):
        _src_wait_shift(r + 1, b ^ 1)
        _g_start(r + 1, b ^ 1)
        _g_wait(r, b)
        _src_start(r + 2, b)
        _scatter(r, b)

    def _step2(i, _):
        _step(2 * i, 0)
        _step(2 * i + 1, 1)
        return 0

    lax.fori_loop(0, (CPT - 3) // 2, _step2, 0)
    r = CPT - 3
    _step(r, 0)
    _src_wait_shift(r + 2, 0)
    _g_start(r + 2, 0)
    _g_wait(r + 1, 1)
    _scatter(r + 1, 1)
    _g_wait(r + 2, 0)
    _scatter(r + 2, 0)

    plsc.subcore_barrier()
    pltpu.sync_copy(acc.at[pl.ds(rbase, RPT)], s_hbm.at[c * NS + s])


@functools.partial(
    pl.kernel,
    out_type=jax.ShapeDtypeStruct((2 * NS, RPT, HH), _f32),
    mesh=_MESH,
    scratch_types=[
        pltpu.VMEM_SHARED((N, HH), _f32),
        pltpu.VMEM((2, CH), jnp.int32),
        pltpu.VMEM((CPT, CH), jnp.int32),
        pltpu.VMEM((2, CH, HH), _f32),
        pltpu.SemaphoreType.DMA,
        pltpu.SemaphoreType.DMA,
        pltpu.SemaphoreType.DMA,
        pltpu.SemaphoreType.DMA,
    ],
)
def _conv_sc(t2_hbm, t23_hbm, src_hbm, dst_hbm, s_hbm, acc, idxs_v, idxd_v,
             rows_v, sem_g0, sem_g1, sem_s0, sem_s1):
    _conv_body(t2_hbm, t23_hbm, src_hbm, dst_hbm, s_hbm, acc, idxs_v, idxd_v,
               rows_v, sem_g0, sem_g1, sem_s0, sem_s1)


def kernel(x, edge_index, edge_attr, ground_node, node_subnode_index,
           subgraph_edge_index, subnode_node_index, batch, params):
    del edge_attr
    sets = (edge_index, node_subnode_index, subgraph_edge_index,
            subnode_node_index)
    srcs = [e[0] for e in sets]
    dsts = [e[1].reshape(NS, CPT, CH) for e in sets]
    dst_all = jnp.concatenate(dsts, axis=0)

    deg16 = _degrees_sc(dst_all)
    degs = deg16.reshape(4, N, DW)[:, :, 0:1]

    gn = ground_node.astype(_f32).reshape(N, 1)
    seg = batch.astype(jnp.int32).reshape(N, 1)

    h = _embed(x, params["embed"][0], params["embed"][1].reshape(1, H))

    layer_names = ("ground", "g2s", "sub", "s2g")
    upgs = (True, False, False, True)
    for i in range(DEPTH):
        for e in range(4):
            w, b = params[layer_names[e]][i]
            t2 = _pre(h, w, degs[e])
            t2f = t2.reshape(2 * N, HH)
            t23 = t2.reshape(2 * NS, RPT, HH)
            sagg = _conv_sc(t2f, t23, srcs[e], dsts[e])
            h = _post(sagg.reshape(2, N, HH), h, gn, degs[e],
                      b.reshape(1, H), upgs[e])

    return _pool_head(h, seg, params["head"][0],
                      params["head"][1].reshape(1, OUT))
